# Initial kernel scaffold; baseline (speedup 1.0000x reference)
import jax
import jax.numpy as jnp
from jax.experimental import pallas as pl


def kernel(x, edge_index, W_pre, b_pre, W1s, b1s, W2s, b2s, W_post, b_post, W_ro, b_ro):
    raise NotImplementedError("write your pallas kernel here")


# speedup vs baseline: 2.7802x; 2.7802x over previous
import functools

import jax
import jax.numpy as jnp
from jax import lax
from jax.experimental import pallas as pl
from jax.experimental.pallas import tpu as pltpu
from jax.experimental.pallas import tpu_sc as plsc

_N = 10000
_E = 320000
_D = 128
_NCORE = 2
_NSUB = 16
_NW = _NCORE * _NSUB
_CH = 128
_NCH = 80
_EPW = _CH * _NCH
_EPAD = _NW * _EPW
_NPAD = 10112
_RPT = _NPAD // _NSUB


def _seg_sum_sc(h, src_r, dst_r):
    mesh = plsc.VectorSubcoreMesh(core_axis_name="c", subcore_axis_name="s")

    @functools.partial(
        pl.kernel,
        mesh=mesh,
        out_type=jax.ShapeDtypeStruct((_NCORE, _NPAD, _D), jnp.float32),
        scratch_types=[
            pltpu.VMEM((_NCH, _CH), jnp.int32),
            pltpu.VMEM((_NCH, _CH), jnp.int32),
            pltpu.VMEM((_CH, _D), jnp.float32),
            pltpu.VMEM_SHARED((_NPAD, _D), jnp.float32),
            pltpu.SemaphoreType.DMA,
        ],
    )
    def seg_kernel(h_hbm, src_hbm, dst_hbm, out_hbm, srcv, dstv, rows, acc, gsem):
        cid = lax.axis_index("c")
        sid = lax.axis_index("s")
        wid = sid * _NCORE + cid

        def zrow(i, carry):
            for j in range(_D // 16):
                rows[i, pl.ds(16 * j, 16)] = jnp.zeros((16,), jnp.float32)
            return carry

        lax.fori_loop(0, _CH, zrow, 0)
        base = sid * _RPT
        for k in range(_RPT // _CH):
            pltpu.sync_copy(rows, acc.at[pl.ds(base + k * _CH, _CH)])
        rem = _RPT % _CH
        if rem:
            pltpu.sync_copy(rows.at[pl.ds(0, rem)],
                            acc.at[pl.ds(base + (_RPT // _CH) * _CH, rem)])
        plsc.subcore_barrier()

        pltpu.sync_copy(src_hbm.at[wid], srcv)
        pltpu.sync_copy(dst_hbm.at[wid], dstv)

        def body(j, carry):
            pltpu.async_copy(h_hbm.at[srcv.at[j]], rows, gsem).wait()
            pltpu.sync_copy(rows, acc.at[dstv.at[j]], add=True)
            return carry

        lax.fori_loop(0, _NCH, body, 0)
        plsc.subcore_barrier()

        pltpu.sync_copy(acc.at[pl.ds(base, _RPT)],
                        out_hbm.at[cid, pl.ds(base, _RPT)])

    return seg_kernel(h, src_r, dst_r)


_BM = 2000


def _full(shape):
    return pl.BlockSpec(shape, lambda i: (0, 0))


def _pre_tc(x, w, b):
    def body(x_ref, w_ref, b_ref, o_ref):
        o_ref[...] = (
            jnp.dot(x_ref[...], w_ref[...], preferred_element_type=jnp.float32)
            + b_ref[...]
        )

    return pl.pallas_call(
        body,
        grid=(_N // _BM,),
        in_specs=[
            pl.BlockSpec((_BM, _D), lambda i: (i, 0)),
            _full((_D, _D)),
            _full((1, _D)),
        ],
        out_specs=pl.BlockSpec((_BM, _D), lambda i: (i, 0)),
        out_shape=jax.ShapeDtypeStruct((_N, _D), jnp.float32),
    )(x, w, b.reshape(1, _D))


def _mlp_tc(h, agg, w1, b1, w2, b2):
    def body(h_ref, a0_ref, a1_ref, w1_ref, b1_ref, w2_ref, b2_ref, o_ref):
        z = h_ref[...] + a0_ref[...] + a1_ref[...]
        z = jnp.maximum(
            jnp.dot(z, w1_ref[...], preferred_element_type=jnp.float32)
            + b1_ref[...],
            0.0,
        )
        z = (
            jnp.dot(z, w2_ref[...], preferred_element_type=jnp.float32)
            + b2_ref[...]
        )
        o_ref[...] = jnp.maximum(z, 0.0)

    return pl.pallas_call(
        body,
        grid=(_N // _BM,),
        in_specs=[
            pl.BlockSpec((_BM, _D), lambda i: (i, 0)),
            pl.BlockSpec((_BM, _D), lambda i: (i, 0)),
            pl.BlockSpec((_BM, _D), lambda i: (i, 0)),
            _full((_D, _D)),
            _full((1, _D)),
            _full((_D, _D)),
            _full((1, _D)),
        ],
        out_specs=pl.BlockSpec((_BM, _D), lambda i: (i, 0)),
        out_shape=jax.ShapeDtypeStruct((_N, _D), jnp.float32),
    )(h, agg[0], agg[1], w1, b1.reshape(1, _D), w2, b2.reshape(1, _D))


def _post_tc(h, wp, bp, wr_pad, br_pad):
    def body(h_ref, wp_ref, bp_ref, wr_ref, br_ref, o_ref):
        t = jnp.maximum(
            jnp.dot(h_ref[...], wp_ref[...], preferred_element_type=jnp.float32)
            + bp_ref[...],
            0.0,
        )
        z = (
            jnp.dot(t, wr_ref[...], preferred_element_type=jnp.float32)
            + br_ref[...]
        )
        m = jnp.max(z, axis=1, keepdims=True)
        lse = jnp.log(jnp.sum(jnp.exp(z - m), axis=1, keepdims=True)) + m
        o_ref[...] = z - lse

    return pl.pallas_call(
        body,
        grid=(_N // _BM,),
        in_specs=[
            pl.BlockSpec((_BM, _D), lambda i: (i, 0)),
            _full((_D, _D)),
            _full((1, _D)),
            _full((_D, _D)),
            _full((1, _D)),
        ],
        out_specs=pl.BlockSpec((_BM, _D), lambda i: (i, 0)),
        out_shape=jax.ShapeDtypeStruct((_N, _D), jnp.float32),
    )(h, wp, bp.reshape(1, _D), wr_pad, br_pad)


def kernel(x, edge_index, W_pre, b_pre, W1s, b1s, W2s, b2s, W_post, b_post,
           W_ro, b_ro):
    src = edge_index[0]
    dst = edge_index[1]
    npad = _EPAD - _E
    src_r = jnp.concatenate([src, jnp.zeros((npad,), jnp.int32)]).reshape(
        _NW, _NCH, _CH)
    dst_r = jnp.concatenate([dst, jnp.full((npad,), _N, jnp.int32)]).reshape(
        _NW, _NCH, _CH)

    h = _pre_tc(x, W_pre, b_pre)
    for l in range(3):
        agg = _seg_sum_sc(h, src_r, dst_r)
        h = _mlp_tc(h, agg, W1s[l], b1s[l], W2s[l], b2s[l])

    nclass = W_ro.shape[1]
    wr_pad = jnp.zeros((_D, _D), jnp.float32).at[:, :nclass].set(W_ro)
    br_pad = jnp.full((1, _D), -1e30, jnp.float32).at[0, :nclass].set(b_ro)
    out = _post_tc(h, W_post, b_post, wr_pad, br_pad)[:, :nclass]
    return (out, h, h)
